# baseline (speedup 1.0000x reference)
import functools

import jax
import jax.numpy as jnp
from jax import lax
from jax.experimental import pallas as pl
from jax.experimental.pallas import tpu as pltpu
from jax.experimental.pallas import tpu_sc as plsc

_C = 2
_RING = 6
_UNROLL = 4


def _make_sc_kernel(B, T, D):
    info = plsc.get_sparse_core_info()
    NC, NS, L = info.num_cores, info.num_subcores, info.num_lanes
    NW = NC * NS
    rows_per_w = T // NW
    n_units = rows_per_w // _C
    mesh = plsc.VectorSubcoreMesh(core_axis_name="c", subcore_axis_name="s")

    @functools.partial(
        pl.kernel,
        mesh=mesh,
        out_type=jax.ShapeDtypeStruct((B, T, D), jnp.float32),
        scratch_types=(
            [pltpu.VMEM((B, _C, D), jnp.float32) for _ in range(_RING)]
            + [pltpu.VMEM((_C, D), jnp.float32) for _ in range(_RING)]
            + [pltpu.SemaphoreType.DMA for _ in range(2 * _RING)]
        ),
    )
    def k(x_hbm, e_hbm, o_hbm,
          xb0, xb1, xb2, xb3, xb4, xb5, eb0, eb1, eb2, eb3, eb4, eb5,
          sl0, sl1, sl2, sl3, sl4, sl5, ss0, ss1, ss2, ss3, ss4, ss5):
        xb = (xb0, xb1, xb2, xb3, xb4, xb5)
        eb = (eb0, eb1, eb2, eb3, eb4, eb5)
        sld = (sl0, sl1, sl2, sl3, sl4, sl5)
        sst = (ss0, ss1, ss2, ss3, ss4, ss5)
        wid = lax.axis_index("s") * NC + lax.axis_index("c")
        t0 = wid * rows_per_w

        def fire_loads(u, r):
            tc = t0 + u * _C
            pltpu.async_copy(e_hbm.at[pl.ds(tc, _C)], eb[r], sld[r])
            pltpu.async_copy(
                x_hbm.at[:, pl.ds(tc, _C)], xb[r], sld[r])

        def drain_loads(r):
            pltpu.make_async_copy(
                x_hbm.at[:, pl.ds(0, _C)], xb[r], sld[r]).wait()
            pltpu.make_async_copy(
                e_hbm.at[pl.ds(0, _C)], eb[r], sld[r]).wait()

        def fire_stores(u, r):
            tc = t0 + u * _C
            pltpu.async_copy(
                xb[r], o_hbm.at[:, pl.ds(tc, _C)], sst[r])

        def drain_stores(r):
            pltpu.make_async_copy(
                xb[r], o_hbm.at[:, pl.ds(0, _C)], sst[r]).wait()

        def compute(r):
            xr, er = xb[r], eb[r]

            @plsc.parallel_loop(0, D, step=L, unroll=_UNROLL)
            def _(col):
                for q in range(_C):
                    e = er[q, pl.ds(col, L)]
                    for b in range(B):
                        xr[b, q, pl.ds(col, L)] = xr[b, q, pl.ds(col, L)] + e

        def unit(u, j, drain_st, fire_ld):
            nxt = (j + 1) % _RING
            if drain_st:
                drain_stores(nxt)
            if fire_ld:
                fire_loads(u + 1, nxt)
            drain_loads(j)
            compute(j)
            fire_stores(u, j)

        fire_loads(0, 0)
        for j in range(_RING):
            unit(j, j, drain_st=(j >= _RING - 1), fire_ld=True)

        def body(kk, _):
            u0 = kk * _RING
            for j in range(_RING):
                unit(u0 + j, j, drain_st=True, fire_ld=True)
            return 0

        lax.fori_loop(1, n_units // _RING, body, 0)

        for u in range((n_units // _RING) * _RING, n_units):
            unit(u, u % _RING, drain_st=True, fire_ld=(u + 1 < n_units))

        for u in range(n_units - (_RING - 1), n_units):
            drain_stores(u % _RING)

    return k


def kernel(x, emb):
    B, T, D = x.shape
    k = _make_sc_kernel(B, T, D)
    return k(x, emb[:T])


# speedup vs baseline: 1.0290x; 1.0290x over previous
import functools

import jax
import jax.numpy as jnp
from jax import lax
from jax.experimental import pallas as pl
from jax.experimental.pallas import tpu as pltpu
from jax.experimental.pallas import tpu_sc as plsc

_C = 2
_RING = 4
_UNROLL = 4


def _make_sc_kernel(B, T, D):
    info = plsc.get_sparse_core_info()
    NC, NS, L = info.num_cores, info.num_subcores, info.num_lanes
    NW = NC * NS
    rows_per_w = T // NW
    n_units = rows_per_w // _C
    mesh = plsc.VectorSubcoreMesh(core_axis_name="c", subcore_axis_name="s")

    @functools.partial(
        pl.kernel,
        mesh=mesh,
        out_type=jax.ShapeDtypeStruct((B, T, D), jnp.float32),
        scratch_types=(
            [pltpu.VMEM((B, _C, D), jnp.float32) for _ in range(_RING)]
            + [pltpu.VMEM((_C, D), jnp.float32) for _ in range(_RING)]
            + [pltpu.SemaphoreType.DMA for _ in range(2 * _RING)]
        ),
    )
    def k(x_hbm, e_hbm, o_hbm, xb0, xb1, xb2, xb3, eb0, eb1, eb2, eb3,
          sl0, sl1, sl2, sl3, ss0, ss1, ss2, ss3):
        xb = (xb0, xb1, xb2, xb3)
        eb = (eb0, eb1, eb2, eb3)
        sld = (sl0, sl1, sl2, sl3)
        sst = (ss0, ss1, ss2, ss3)
        wid = lax.axis_index("s") * NC + lax.axis_index("c")
        t0 = wid * rows_per_w

        def fire_loads(u, r):
            tc = t0 + u * _C
            pltpu.async_copy(e_hbm.at[pl.ds(tc, _C)], eb[r], sld[r])
            pltpu.async_copy(
                x_hbm.at[:, pl.ds(tc, _C)], xb[r], sld[r])

        def drain_loads(r):
            pltpu.make_async_copy(
                x_hbm.at[:, pl.ds(0, _C)], xb[r], sld[r]).wait()
            pltpu.make_async_copy(
                e_hbm.at[pl.ds(0, _C)], eb[r], sld[r]).wait()

        def fire_stores(u, r):
            tc = t0 + u * _C
            pltpu.async_copy(
                xb[r], o_hbm.at[:, pl.ds(tc, _C)], sst[r])

        def drain_stores(r):
            pltpu.make_async_copy(
                xb[r], o_hbm.at[:, pl.ds(0, _C)], sst[r]).wait()

        def compute(r):
            xr, er = xb[r], eb[r]

            @plsc.parallel_loop(0, D, step=L, unroll=_UNROLL)
            def _(col):
                for q in range(_C):
                    e = er[q, pl.ds(col, L)]
                    for b in range(B):
                        xr[b, q, pl.ds(col, L)] = xr[b, q, pl.ds(col, L)] + e

        def unit(u, j, drain_st, fire_ld):
            nxt = (j + 1) % _RING
            if drain_st:
                drain_stores(nxt)
            if fire_ld:
                fire_loads(u + 1, nxt)
            drain_loads(j)
            compute(j)
            fire_stores(u, j)

        fire_loads(0, 0)
        for j in range(_RING):
            unit(j, j, drain_st=(j >= _RING - 1), fire_ld=True)

        def body(kk, _):
            u0 = kk * _RING
            for j in range(_RING):
                unit(u0 + j, j, drain_st=True, fire_ld=True)
            return 0

        lax.fori_loop(1, n_units // _RING, body, 0)

        for u in range((n_units // _RING) * _RING, n_units):
            unit(u, u % _RING, drain_st=True, fire_ld=(u + 1 < n_units))

        for u in range(n_units - (_RING - 1), n_units):
            drain_stores(u % _RING)

    return k


def kernel(x, emb):
    B, T, D = x.shape
    k = _make_sc_kernel(B, T, D)
    return k(x, emb[:T])
